# Initial kernel scaffold; baseline (speedup 1.0000x reference)
import functools

import jax
import jax.numpy as jnp
from jax import lax
from jax.experimental import pallas as pl
from jax.experimental.pallas import tpu as pltpu
from jax.experimental.pallas import tpu_sc as plsc

N = 10000
E = 320000
D = 128
NC = 2
NS = 16
NW = NC * NS
EPT = E // NW
C = 80
NCH = EPT // C
NP = 10240
SLAB = NP // NS
L = 16


def _sc_body(node_h, src_h, dst_h, attr_h, sums_h, cnts_h,
             acc_sh, cntall_sh, src_v, dst_v, attr_v, rows_v, zrow_v,
             cnt_v, tmp_v, tacc_v, sem):
    c = lax.axis_index("c")
    s = lax.axis_index("s")
    wid = c * NS + s
    zero16 = jnp.zeros((L,), jnp.float32)
    ones16 = jnp.ones((L,), jnp.float32)

    def _zrow(i, _):
        for j in range(D // L):
            zrow_v[i, pl.ds(j * L, L)] = zero16
        return 0
    lax.fori_loop(0, 128, _zrow, 0)

    def _zcnt(i, _):
        cnt_v[pl.ds(i * L, L)] = zero16
        return 0
    lax.fori_loop(0, NP // L, _zcnt, 0)

    for k in range(SLAB // 128):
        pltpu.sync_copy(zrow_v, acc_sh.at[pl.ds(s * SLAB + k * 128, 128)])
    plsc.subcore_barrier()

    def _chunk(g, _):
        pltpu.sync_copy(src_h.at[wid, g], src_v)
        pltpu.sync_copy(dst_h.at[wid, g], dst_v)
        pltpu.sync_copy(attr_h.at[wid, g], attr_v)
        pltpu.async_copy(node_h.at[src_v], rows_v, sem).wait()

        def _row(e, _):
            a = plsc.load_gather(attr_v, [jnp.full((L,), e, jnp.int32)])
            for j in range(D // L):
                sl = pl.ds(j * L, L)
                rows_v[e, sl] = rows_v[e, sl] * a
            return 0
        lax.fori_loop(0, C, _row, 0)

        for k in range(C // L):
            d16 = dst_v[pl.ds(k * L, L)]
            plsc.addupdate_scatter(cnt_v, [d16], ones16)

        pltpu.sync_copy(rows_v, acc_sh.at[dst_v], add=True)
        return 0
    lax.fori_loop(0, NCH, _chunk, 0)
    plsc.subcore_barrier()

    pltpu.sync_copy(acc_sh.at[pl.ds(s * SLAB, SLAB)],
                    sums_h.at[c, pl.ds(s * SLAB, SLAB)])

    pltpu.sync_copy(cnt_v, cntall_sh.at[s])
    plsc.subcore_barrier()

    def _ztacc(i, _):
        tacc_v[pl.ds(i * L, L)] = zero16
        return 0
    lax.fori_loop(0, SLAB // L, _ztacc, 0)

    def _red(t, _):
        pltpu.sync_copy(cntall_sh.at[t, pl.ds(s * SLAB, SLAB)], tmp_v)

        def _add(k, _):
            sl = pl.ds(k * L, L)
            tacc_v[sl] = tacc_v[sl] + tmp_v[sl]
            return 0
        lax.fori_loop(0, SLAB // L, _add, 0)
        return 0
    lax.fori_loop(0, NS, _red, 0)
    pltpu.sync_copy(tacc_v, cnts_h.at[c, pl.ds(s * SLAB, SLAB)])


@jax.jit
def _sc_aggregate(node, src, dst, attr):
    mesh = plsc.VectorSubcoreMesh(core_axis_name="c", subcore_axis_name="s",
                                  num_cores=NC, num_subcores=NS)
    f = pl.kernel(
        _sc_body,
        out_type=[jax.ShapeDtypeStruct((NC, NP, D), jnp.float32),
                  jax.ShapeDtypeStruct((NC, NP), jnp.float32)],
        mesh=mesh,
        scratch_types=[
            pltpu.VMEM_SHARED((NP, D), jnp.float32),
            pltpu.VMEM_SHARED((NS, NP), jnp.float32),
            pltpu.VMEM((C,), jnp.int32),
            pltpu.VMEM((C,), jnp.int32),
            pltpu.VMEM((C,), jnp.float32),
            pltpu.VMEM((C, D), jnp.float32),
            pltpu.VMEM((128, D), jnp.float32),
            pltpu.VMEM((NP,), jnp.float32),
            pltpu.VMEM((SLAB,), jnp.float32),
            pltpu.VMEM((SLAB,), jnp.float32),
            pltpu.SemaphoreType.DMA,
        ],
    )
    return f(node, src, dst, attr)


def _tc_body(sums_ref, cnts_ref, node_ref, wrel_ref, wroot_ref, w1_ref,
             w2_ref, brel_ref, b1_ref, b2_ref, ln1w_ref, ln1b_ref,
             ln2w_ref, ln2b_ref, out_ref):
    dn = (((1,), (1,)), ((), ()))
    cnt = jnp.clip(cnts_ref[0] + cnts_ref[1], 1.0, None)
    agg = (sums_ref[0] + sums_ref[1]) / cnt
    h = (lax.dot_general(agg, wrel_ref[...], dn,
                         preferred_element_type=jnp.float32)
         + brel_ref[...]
         + lax.dot_general(node_ref[...], wroot_ref[...], dn,
                           preferred_element_type=jnp.float32))

    def _ln_relu(t, w, b):
        mu = jnp.mean(t, axis=-1, keepdims=True)
        d = t - mu
        var = jnp.mean(d * d, axis=-1, keepdims=True)
        return jnp.maximum(d * lax.rsqrt(var + 1e-5) * w + b, 0.0)

    t1 = lax.dot_general(h, w1_ref[...], dn,
                         preferred_element_type=jnp.float32) + b1_ref[...]
    y1 = _ln_relu(t1, ln1w_ref[...], ln1b_ref[...])
    t2 = lax.dot_general(y1, w2_ref[...], dn,
                         preferred_element_type=jnp.float32) + b2_ref[...]
    out_ref[...] = _ln_relu(t2, ln2w_ref[...], ln2b_ref[...])


BR = 1024


@jax.jit
def _tc_dense(sums, cnts, node_p, W_rel, W_root, W1, W2,
              b_rel, b1, b2, ln1_w, ln1_b, ln2_w, ln2_b):
    full = pl.BlockSpec((D, D), lambda i: (0, 0))
    vec = pl.BlockSpec((1, D), lambda i: (0, 0))
    return pl.pallas_call(
        _tc_body,
        grid=(NP // BR,),
        in_specs=[
            pl.BlockSpec((NC, BR, D), lambda i: (0, i, 0)),
            pl.BlockSpec((NC, BR, 1), lambda i: (0, i, 0)),
            pl.BlockSpec((BR, D), lambda i: (i, 0)),
            full, full, full, full, vec, vec, vec, vec, vec, vec, vec,
        ],
        out_specs=pl.BlockSpec((BR, D), lambda i: (i, 0)),
        out_shape=jax.ShapeDtypeStruct((NP, D), jnp.float32),
    )(sums, cnts.reshape(NC, NP, 1), node_p, W_rel, W_root, W1, W2,
      b_rel.reshape(1, D), b1.reshape(1, D), b2.reshape(1, D),
      ln1_w.reshape(1, D), ln1_b.reshape(1, D),
      ln2_w.reshape(1, D), ln2_b.reshape(1, D))


def kernel(node, edge_index, edge_attr, batch_ptr, W_rel, b_rel, W_root,
           W1, b1, W2, b2, ln1_w, ln1_b, ln2_w, ln2_b):
    src = edge_index[0].reshape(NW, NCH, C)
    dst = edge_index[1].reshape(NW, NCH, C)
    attr = edge_attr.reshape(NW, NCH, C)
    sums, cnts = _sc_aggregate(node, src, dst, attr)
    node_p = jnp.pad(node, ((0, NP - N), (0, 0)))
    out = _tc_dense(sums, cnts, node_p, W_rel, W_root, W1, W2,
                    b_rel, b1, b2, ln1_w, ln1_b, ln2_w, ln2_b)
    return out[:N]


# speedup vs baseline: 4.5304x; 4.5304x over previous
import functools

import jax
import jax.numpy as jnp
from jax import lax
from jax.experimental import pallas as pl
from jax.experimental.pallas import tpu as pltpu
from jax.experimental.pallas import tpu_sc as plsc

N = 10000
E = 320000
D = 128
NC = 2
NS = 16
NW = NC * NS
EPT = E // NW
C = 80
NCH = EPT // C
NP = 10240
SLAB = NP // NS
L = 16


def _bcast_lane(v16, i):
    idx = jnp.full((L,), i, jnp.int32)
    return lax.gather(
        v16, idx[:, None],
        lax.GatherDimensionNumbers(offset_dims=(), collapsed_slice_dims=(0,),
                                   start_index_map=(0,)),
        (1,), mode=lax.GatherScatterMode.PROMISE_IN_BOUNDS)


def _sc_body(node_h, src_h, dst_h, attr_h, sums_h, cnts_h,
             acc_sh, cntall_sh, src_v, dst_v, attr_v, rows_v, zrow_v,
             cnt_v, tmp_v, tacc_v, sem):
    c = lax.axis_index("c")
    s = lax.axis_index("s")
    wid = c * NS + s
    zero16 = jnp.zeros((L,), jnp.float32)
    ones16 = jnp.ones((L,), jnp.float32)

    def _zrow(i, _):
        for j in range(D // L):
            zrow_v[i, pl.ds(j * L, L)] = zero16
        return 0
    lax.fori_loop(0, 128, _zrow, 0)

    def _zcnt(i, _):
        cnt_v[pl.ds(i * L, L)] = zero16
        return 0
    lax.fori_loop(0, NP // L, _zcnt, 0)

    for k in range(SLAB // 128):
        pltpu.sync_copy(zrow_v, acc_sh.at[pl.ds(s * SLAB + k * 128, 128)])
    plsc.subcore_barrier()

    def _chunk(g, _):
        pltpu.sync_copy(src_h.at[wid, g], src_v)
        pltpu.sync_copy(dst_h.at[wid, g], dst_v)
        pltpu.sync_copy(attr_h.at[wid, g], attr_v)
        pltpu.async_copy(node_h.at[src_v], rows_v, sem).wait()

        def _grp(k, _):
            a16 = attr_v[pl.ds(k * L, L)]
            for i in range(L):
                a = _bcast_lane(a16, i)
                e = k * L + i
                for j in range(D // L):
                    sl = pl.ds(j * L, L)
                    rows_v[e, sl] = rows_v[e, sl] * a
            return 0
        lax.fori_loop(0, C // L, _grp, 0)

        for k in range(C // L):
            d16 = dst_v[pl.ds(k * L, L)]
            plsc.addupdate_scatter(cnt_v, [d16], ones16)

        pltpu.sync_copy(rows_v, acc_sh.at[dst_v], add=True)
        return 0
    lax.fori_loop(0, NCH, _chunk, 0)
    plsc.subcore_barrier()

    pltpu.sync_copy(acc_sh.at[pl.ds(s * SLAB, SLAB)],
                    sums_h.at[c, pl.ds(s * SLAB, SLAB)])

    pltpu.sync_copy(cnt_v, cntall_sh.at[s])
    plsc.subcore_barrier()

    def _ztacc(i, _):
        tacc_v[pl.ds(i * L, L)] = zero16
        return 0
    lax.fori_loop(0, SLAB // L, _ztacc, 0)

    def _red(t, _):
        pltpu.sync_copy(cntall_sh.at[t, pl.ds(s * SLAB, SLAB)], tmp_v)

        def _add(k, _):
            sl = pl.ds(k * L, L)
            tacc_v[sl] = tacc_v[sl] + tmp_v[sl]
            return 0
        lax.fori_loop(0, SLAB // L, _add, 0)
        return 0
    lax.fori_loop(0, NS, _red, 0)
    pltpu.sync_copy(tacc_v, cnts_h.at[c, pl.ds(s * SLAB, SLAB)])


@jax.jit
def _sc_aggregate(node, src, dst, attr):
    mesh = plsc.VectorSubcoreMesh(core_axis_name="c", subcore_axis_name="s",
                                  num_cores=NC, num_subcores=NS)
    f = pl.kernel(
        _sc_body,
        out_type=[jax.ShapeDtypeStruct((NC, NP, D), jnp.float32),
                  jax.ShapeDtypeStruct((NC, NP), jnp.float32)],
        mesh=mesh,
        compiler_params=pltpu.CompilerParams(needs_layout_passes=False),
        scratch_types=[
            pltpu.VMEM_SHARED((NP, D), jnp.float32),
            pltpu.VMEM_SHARED((NS, NP), jnp.float32),
            pltpu.VMEM((C,), jnp.int32),
            pltpu.VMEM((C,), jnp.int32),
            pltpu.VMEM((C,), jnp.float32),
            pltpu.VMEM((C, D), jnp.float32),
            pltpu.VMEM((128, D), jnp.float32),
            pltpu.VMEM((NP,), jnp.float32),
            pltpu.VMEM((SLAB,), jnp.float32),
            pltpu.VMEM((SLAB,), jnp.float32),
            pltpu.SemaphoreType.DMA,
        ],
    )
    return f(node, src, dst, attr)


def _tc_body(sums_ref, cnts_ref, node_ref, wrel_ref, wroot_ref, w1_ref,
             w2_ref, brel_ref, b1_ref, b2_ref, ln1w_ref, ln1b_ref,
             ln2w_ref, ln2b_ref, out_ref):
    dn = (((1,), (1,)), ((), ()))
    cnt = jnp.clip(cnts_ref[0] + cnts_ref[1], 1.0, None)
    agg = (sums_ref[0] + sums_ref[1]) / cnt
    h = (lax.dot_general(agg, wrel_ref[...], dn,
                         preferred_element_type=jnp.float32)
         + brel_ref[...]
         + lax.dot_general(node_ref[...], wroot_ref[...], dn,
                           preferred_element_type=jnp.float32))

    def _ln_relu(t, w, b):
        mu = jnp.mean(t, axis=-1, keepdims=True)
        d = t - mu
        var = jnp.mean(d * d, axis=-1, keepdims=True)
        return jnp.maximum(d * lax.rsqrt(var + 1e-5) * w + b, 0.0)

    t1 = lax.dot_general(h, w1_ref[...], dn,
                         preferred_element_type=jnp.float32) + b1_ref[...]
    y1 = _ln_relu(t1, ln1w_ref[...], ln1b_ref[...])
    t2 = lax.dot_general(y1, w2_ref[...], dn,
                         preferred_element_type=jnp.float32) + b2_ref[...]
    out_ref[...] = _ln_relu(t2, ln2w_ref[...], ln2b_ref[...])


BR = 1024


@jax.jit
def _tc_dense(sums, cnts, node_p, W_rel, W_root, W1, W2,
              b_rel, b1, b2, ln1_w, ln1_b, ln2_w, ln2_b):
    full = pl.BlockSpec((D, D), lambda i: (0, 0))
    vec = pl.BlockSpec((1, D), lambda i: (0, 0))
    return pl.pallas_call(
        _tc_body,
        grid=(NP // BR,),
        in_specs=[
            pl.BlockSpec((NC, BR, D), lambda i: (0, i, 0)),
            pl.BlockSpec((NC, BR, 1), lambda i: (0, i, 0)),
            pl.BlockSpec((BR, D), lambda i: (i, 0)),
            full, full, full, full, vec, vec, vec, vec, vec, vec, vec,
        ],
        out_specs=pl.BlockSpec((BR, D), lambda i: (i, 0)),
        out_shape=jax.ShapeDtypeStruct((NP, D), jnp.float32),
    )(sums, cnts.reshape(NC, NP, 1), node_p, W_rel, W_root, W1, W2,
      b_rel.reshape(1, D), b1.reshape(1, D), b2.reshape(1, D),
      ln1_w.reshape(1, D), ln1_b.reshape(1, D),
      ln2_w.reshape(1, D), ln2_b.reshape(1, D))


def kernel(node, edge_index, edge_attr, batch_ptr, W_rel, b_rel, W_root,
           W1, b1, W2, b2, ln1_w, ln1_b, ln2_w, ln2_b):
    src = edge_index[0].reshape(NW, NCH, C)
    dst = edge_index[1].reshape(NW, NCH, C)
    attr = edge_attr.reshape(NW, NCH, C)
    sums, cnts = _sc_aggregate(node, src, dst, attr)
    node_p = jnp.pad(node, ((0, NP - N), (0, 0)))
    out = _tc_dense(sums, cnts, node_p, W_rel, W_root, W1, W2,
                    b_rel, b1, b2, ln1_w, ln1_b, ln2_w, ln2_b)
    return out[:N]
